# speedup vs baseline: 1.6891x; 1.6891x over previous
import jax
import jax.numpy as jnp
from jax import lax
from jax.experimental import pallas as pl
from jax.experimental.pallas import tpu as pltpu
from jax.experimental.pallas import tpu_sc as plsc

_B, _T, _D = 4, 8192, 768
_L = 8192

_INFO = plsc.get_sparse_core_info()
_NC, _NS, _LANES = _INFO.num_cores, _INFO.num_subcores, _INFO.num_lanes
_NW = _NC * _NS
_ROWS = _B * _L
_RPW = _ROWS // _NW
_WPB = _NW // _B
_CHUNK = 128
_NCHUNK = _RPW // _CHUNK


def _gather_body(in_hbm, idx_hbm, out_hbm, idx_v, rows_v, sem):
    wid = lax.axis_index("s") * _NC + lax.axis_index("c")
    base = wid * _RPW
    boff = (wid // _WPB) * _T

    pltpu.sync_copy(idx_hbm.at[pl.ds(base, _RPW)], idx_v)

    def fix(i, carry):
        sl = pl.ds(i * _LANES, _LANES)
        v = idx_v[sl]
        idx_v[sl] = jnp.maximum(v, 0) + boff
        return carry

    lax.fori_loop(0, _RPW // _LANES, fix, 0)

    def chunk(c, carry):
        cbase = c * _CHUNK
        pltpu.async_copy(
            in_hbm.at[idx_v.at[pl.ds(cbase, _CHUNK)]], rows_v, sem
        ).wait()
        pltpu.sync_copy(rows_v, out_hbm.at[pl.ds(base + cbase, _CHUNK)])
        return carry

    lax.fori_loop(0, _NCHUNK, chunk, 0)


@jax.jit
def _gather(flat_in, flat_idx):
    mesh = plsc.VectorSubcoreMesh(core_axis_name="c", subcore_axis_name="s")
    return pl.kernel(
        _gather_body,
        out_type=jax.ShapeDtypeStruct((_ROWS, _D), jnp.float32),
        mesh=mesh,
        scratch_types=[
            pltpu.VMEM((_RPW,), jnp.int32),
            pltpu.VMEM((_CHUNK, _D), jnp.float32),
            pltpu.SemaphoreType.DMA,
        ],
    )(flat_in, flat_idx)


def kernel(inputs, word_ids):
    flat_in = inputs.reshape(_B * _T, _D)
    flat_idx = word_ids.astype(jnp.int32).reshape(_ROWS)
    out = _gather(flat_in, flat_idx)
    return out.reshape(_B, _L, _D)


# speedup vs baseline: 1.7210x; 1.0189x over previous
import jax
import jax.numpy as jnp
from jax import lax
from jax.experimental import pallas as pl
from jax.experimental.pallas import tpu as pltpu
from jax.experimental.pallas import tpu_sc as plsc

_B, _T, _D = 4, 8192, 768
_L = 8192

_INFO = plsc.get_sparse_core_info()
_NC, _NS, _LANES = _INFO.num_cores, _INFO.num_subcores, _INFO.num_lanes
_NW = _NC * _NS
_ROWS = _B * _L
_RPW = _ROWS // _NW
_WPB = _NW // _B
_CHUNK = 64
_NCHUNK = _RPW // _CHUNK


def _gather_body(in_hbm, idx_hbm, out_hbm, idx_v, buf0, buf1,
                 gsem0, gsem1, ssem0, ssem1):
    wid = lax.axis_index("s") * _NC + lax.axis_index("c")
    base = wid * _RPW
    boff = (wid // _WPB) * _T
    bufs = (buf0, buf1)
    gsems = (gsem0, gsem1)
    ssems = (ssem0, ssem1)

    pltpu.sync_copy(idx_hbm.at[pl.ds(base, _RPW)], idx_v)

    def fix(i, carry):
        sl = pl.ds(i * _LANES, _LANES)
        v = idx_v[sl]
        idx_v[sl] = jnp.maximum(v, 0) + boff
        return carry

    lax.fori_loop(0, _RPW // _LANES, fix, 0)

    def gather(c, b):
        return pltpu.make_async_copy(
            in_hbm.at[idx_v.at[pl.ds(c * _CHUNK, _CHUNK)]], bufs[b], gsems[b]
        )

    def scatter(c, b):
        return pltpu.make_async_copy(
            bufs[b], out_hbm.at[pl.ds(base + c * _CHUNK, _CHUNK)], ssems[b]
        )

    gather(0, 0).start()

    def pair(c2, carry):
        c0 = c2 * 2

        @pl.when(c2 >= 1)
        def _():
            scatter(c0 - 1, 1).wait()

        gather(c0 + 1, 1).start()
        gather(c0, 0).wait()
        scatter(c0, 0).start()

        @pl.when(c2 < _NCHUNK // 2 - 1)
        def _():
            scatter(c0, 0).wait()
            gather(c0 + 2, 0).start()

        gather(c0 + 1, 1).wait()
        scatter(c0 + 1, 1).start()
        return carry

    lax.fori_loop(0, _NCHUNK // 2, pair, 0)
    scatter(_NCHUNK - 2, 0).wait()
    scatter(_NCHUNK - 1, 1).wait()


@jax.jit
def _gather(flat_in, flat_idx):
    mesh = plsc.VectorSubcoreMesh(core_axis_name="c", subcore_axis_name="s")
    return pl.kernel(
        _gather_body,
        out_type=jax.ShapeDtypeStruct((_ROWS, _D), jnp.float32),
        mesh=mesh,
        scratch_types=[
            pltpu.VMEM((_RPW,), jnp.int32),
            pltpu.VMEM((_CHUNK, _D), jnp.float32),
            pltpu.VMEM((_CHUNK, _D), jnp.float32),
            pltpu.SemaphoreType.DMA,
            pltpu.SemaphoreType.DMA,
            pltpu.SemaphoreType.DMA,
            pltpu.SemaphoreType.DMA,
        ],
    )(flat_in, flat_idx)


def kernel(inputs, word_ids):
    flat_in = inputs.reshape(_B * _T, _D)
    flat_idx = word_ids.astype(jnp.int32).reshape(_ROWS)
    out = _gather(flat_in, flat_idx)
    return out.reshape(_B, _L, _D)


# speedup vs baseline: 1.7212x; 1.0001x over previous
import jax
import jax.numpy as jnp
from jax import lax
from jax.experimental import pallas as pl
from jax.experimental.pallas import tpu as pltpu
from jax.experimental.pallas import tpu_sc as plsc

_B, _T, _D = 4, 8192, 768
_L = 8192

_INFO = plsc.get_sparse_core_info()
_NC, _NS, _LANES = _INFO.num_cores, _INFO.num_subcores, _INFO.num_lanes
_NW = _NC * _NS
_ROWS = _B * _L
_RPW = _ROWS // _NW
_WPB = _NW // _B
_CHUNK = 32
_NCHUNK = _RPW // _CHUNK
_NBUF = 4


def _gather_body(in_hbm, idx_hbm, out_hbm, idx_v, bufs, gsems, ssems):
    wid = lax.axis_index("s") * _NC + lax.axis_index("c")
    base = wid * _RPW
    boff = (wid // _WPB) * _T

    pltpu.sync_copy(idx_hbm.at[pl.ds(base, _RPW)], idx_v)

    def fix(i, carry):
        sl = pl.ds(i * _LANES, _LANES)
        v = idx_v[sl]
        idx_v[sl] = jnp.maximum(v, 0) + boff
        return carry

    lax.fori_loop(0, _RPW // _LANES, fix, 0)

    def gather(c, b):
        return pltpu.make_async_copy(
            in_hbm.at[idx_v.at[pl.ds(c * _CHUNK, _CHUNK)]], bufs[b], gsems[b]
        )

    def scatter(c, b):
        return pltpu.make_async_copy(
            bufs[b], out_hbm.at[pl.ds(base + c * _CHUNK, _CHUNK)], ssems[b]
        )

    gather(0, 0).start()

    def group(g, carry):
        for b in range(_NBUF):
            c = g * _NBUF + b
            nb = (b + 1) % _NBUF

            @pl.when(c + 1 < _NCHUNK)
            def _():
                @pl.when(c + 1 - _NBUF >= 0)
                def _():
                    scatter(c + 1 - _NBUF, nb).wait()

                gather(c + 1, nb).start()

            gather(c, b).wait()
            scatter(c, b).start()
        return carry

    lax.fori_loop(0, _NCHUNK // _NBUF, group, 0)
    for c in range(_NCHUNK - _NBUF, _NCHUNK):
        scatter(c, c % _NBUF).wait()


@jax.jit
def _gather(flat_in, flat_idx):
    mesh = plsc.VectorSubcoreMesh(core_axis_name="c", subcore_axis_name="s")
    return pl.kernel(
        _gather_body,
        out_type=jax.ShapeDtypeStruct((_ROWS, _D), jnp.float32),
        mesh=mesh,
        scratch_types=[
            pltpu.VMEM((_RPW,), jnp.int32),
            [pltpu.VMEM((_CHUNK, _D), jnp.float32) for _ in range(_NBUF)],
            [pltpu.SemaphoreType.DMA for _ in range(_NBUF)],
            [pltpu.SemaphoreType.DMA for _ in range(_NBUF)],
        ],
    )(flat_in, flat_idx)


def kernel(inputs, word_ids):
    flat_in = inputs.reshape(_B * _T, _D)
    flat_idx = word_ids.astype(jnp.int32).reshape(_ROWS)
    out = _gather(flat_in, flat_idx)
    return out.reshape(_B, _L, _D)


# speedup vs baseline: 1.7471x; 1.0150x over previous
import jax
import jax.numpy as jnp
from jax import lax
from jax.experimental import pallas as pl
from jax.experimental.pallas import tpu as pltpu
from jax.experimental.pallas import tpu_sc as plsc

_B, _T, _D = 4, 8192, 768
_L = 8192

_INFO = plsc.get_sparse_core_info()
_NC, _NS, _LANES = _INFO.num_cores, _INFO.num_subcores, _INFO.num_lanes
_NW = _NC * _NS
_ROWS = _B * _L
_RPW = _ROWS // _NW
_WPB = _NW // _B
_CHUNK = 16
_NCHUNK = _RPW // _CHUNK
_NBUF = 8
_PF = 4


def _gather_body(in_hbm, idx_hbm, out_hbm, idx_v, bufs, gsems, ssems):
    wid = lax.axis_index("s") * _NC + lax.axis_index("c")
    base = wid * _RPW
    boff = (wid // _WPB) * _T

    pltpu.sync_copy(idx_hbm.at[pl.ds(base, _RPW)], idx_v)

    def fix(i, carry):
        sl = pl.ds(i * _LANES, _LANES)
        v = idx_v[sl]
        idx_v[sl] = jnp.maximum(v, 0) + boff
        return carry

    lax.fori_loop(0, _RPW // _LANES, fix, 0)

    def gather(c, b):
        return pltpu.make_async_copy(
            in_hbm.at[idx_v.at[pl.ds(c * _CHUNK, _CHUNK)]], bufs[b], gsems[b]
        )

    def scatter(c, b):
        return pltpu.make_async_copy(
            bufs[b], out_hbm.at[pl.ds(base + c * _CHUNK, _CHUNK)], ssems[b]
        )

    for p in range(_PF):
        gather(p, p).start()

    def group(g, carry):
        for b in range(_NBUF):
            c = g * _NBUF + b
            p = c + _PF
            pb = (b + _PF) % _NBUF

            @pl.when(p < _NCHUNK)
            def _():
                @pl.when(p - _NBUF >= 0)
                def _():
                    scatter(p - _NBUF, pb).wait()

                gather(p, pb).start()

            gather(c, b).wait()
            scatter(c, b).start()
        return carry

    lax.fori_loop(0, _NCHUNK // _NBUF, group, 0)
    for c in range(_NCHUNK - _NBUF, _NCHUNK):
        scatter(c, c % _NBUF).wait()


@jax.jit
def _gather(flat_in, flat_idx):
    mesh = plsc.VectorSubcoreMesh(core_axis_name="c", subcore_axis_name="s")
    return pl.kernel(
        _gather_body,
        out_type=jax.ShapeDtypeStruct((_ROWS, _D), jnp.float32),
        mesh=mesh,
        scratch_types=[
            pltpu.VMEM((_RPW,), jnp.int32),
            [pltpu.VMEM((_CHUNK, _D), jnp.float32) for _ in range(_NBUF)],
            [pltpu.SemaphoreType.DMA for _ in range(_NBUF)],
            [pltpu.SemaphoreType.DMA for _ in range(_NBUF)],
        ],
    )(flat_in, flat_idx)


def kernel(inputs, word_ids):
    flat_in = inputs.reshape(_B * _T, _D)
    flat_idx = word_ids.astype(jnp.int32).reshape(_ROWS)
    out = _gather(flat_in, flat_idx)
    return out.reshape(_B, _L, _D)


# speedup vs baseline: 1.7491x; 1.0011x over previous
import jax
import jax.numpy as jnp
from jax import lax
from jax.experimental import pallas as pl
from jax.experimental.pallas import tpu as pltpu
from jax.experimental.pallas import tpu_sc as plsc

_B, _T, _D = 4, 8192, 768
_L = 8192

_INFO = plsc.get_sparse_core_info()
_NC, _NS, _LANES = _INFO.num_cores, _INFO.num_subcores, _INFO.num_lanes
_NW = _NC * _NS
_ROWS = _B * _L
_RPW = _ROWS // _NW
_WPB = _NW // _B
_CHUNK = 16
_NCHUNK = _RPW // _CHUNK
_NBUF = 8
_PF = 4


def _gather_body(in_hbm, idx_hbm, out_hbm, idx_v, bufs, gsems, ssems):
    wid = lax.axis_index("s") * _NC + lax.axis_index("c")
    base = wid * _RPW
    boff = (wid // _WPB) * _T

    pltpu.sync_copy(idx_hbm.at[pl.ds(base, _RPW)], idx_v)

    def fix(c):
        sl = pl.ds(c * _CHUNK, _CHUNK)
        idx_v[sl] = jnp.maximum(idx_v[sl], 0) + boff

    def gather(c, b):
        return pltpu.make_async_copy(
            in_hbm.at[idx_v.at[pl.ds(c * _CHUNK, _CHUNK)]], bufs[b], gsems[b]
        )

    def scatter(c, b):
        return pltpu.make_async_copy(
            bufs[b], out_hbm.at[pl.ds(base + c * _CHUNK, _CHUNK)], ssems[b]
        )

    for p in range(_PF):
        fix(p)
        gather(p, p).start()

    def group(g, carry):
        for b in range(_NBUF):
            c = g * _NBUF + b
            p = c + _PF
            pb = (b + _PF) % _NBUF

            @pl.when(p < _NCHUNK)
            def _():
                @pl.when(p - _NBUF >= 0)
                def _():
                    scatter(p - _NBUF, pb).wait()

                fix(p)
                gather(p, pb).start()

            gather(c, b).wait()
            scatter(c, b).start()
        return carry

    lax.fori_loop(0, _NCHUNK // _NBUF, group, 0)
    for c in range(_NCHUNK - _NBUF, _NCHUNK):
        scatter(c, c % _NBUF).wait()


@jax.jit
def _gather(flat_in, flat_idx):
    mesh = plsc.VectorSubcoreMesh(core_axis_name="c", subcore_axis_name="s")
    return pl.kernel(
        _gather_body,
        out_type=jax.ShapeDtypeStruct((_ROWS, _D), jnp.float32),
        mesh=mesh,
        scratch_types=[
            pltpu.VMEM((_RPW,), jnp.int32),
            [pltpu.VMEM((_CHUNK, _D), jnp.float32) for _ in range(_NBUF)],
            [pltpu.SemaphoreType.DMA for _ in range(_NBUF)],
            [pltpu.SemaphoreType.DMA for _ in range(_NBUF)],
        ],
    )(flat_in, flat_idx)


def kernel(inputs, word_ids):
    flat_in = inputs.reshape(_B * _T, _D)
    flat_idx = word_ids.astype(jnp.int32).reshape(_ROWS)
    out = _gather(flat_in, flat_idx)
    return out.reshape(_B, _L, _D)


# speedup vs baseline: 1.7540x; 1.0028x over previous
import jax
import jax.numpy as jnp
from jax import lax
from jax.experimental import pallas as pl
from jax.experimental.pallas import tpu as pltpu
from jax.experimental.pallas import tpu_sc as plsc

_B, _T, _D = 4, 8192, 768
_L = 8192

_INFO = plsc.get_sparse_core_info()
_NC, _NS, _LANES = _INFO.num_cores, _INFO.num_subcores, _INFO.num_lanes
_NW = _NC * _NS
_ROWS = _B * _L
_RPW = _ROWS // _NW
_WPB = _NW // _B
_CHUNK = 16
_NCHUNK = _RPW // _CHUNK
_NBUF = 8
_PF = 4


def _gather_body(in_hbm, idx_hbm, out_hbm, idx_v, bufs, gsems, ssems):
    wid = lax.axis_index("s") * _NC + lax.axis_index("c")
    base = wid * _RPW
    boff = (wid // _WPB) * _T

    pltpu.sync_copy(
        idx_hbm.at[wid // _WPB, pl.ds((wid % _WPB) * _RPW, _RPW)], idx_v
    )

    def fix(c):
        sl = pl.ds(c * _CHUNK, _CHUNK)
        idx_v[sl] = jnp.maximum(idx_v[sl], 0) + boff

    def gather(c, b):
        return pltpu.make_async_copy(
            in_hbm.at[idx_v.at[pl.ds(c * _CHUNK, _CHUNK)]], bufs[b], gsems[b]
        )

    def scatter(c, b):
        return pltpu.make_async_copy(
            bufs[b], out_hbm.at[pl.ds(base + c * _CHUNK, _CHUNK)], ssems[b]
        )

    for p in range(_PF):
        fix(p)
        gather(p, p).start()

    def group(g, carry):
        for b in range(_NBUF):
            c = g * _NBUF + b
            p = c + _PF
            pb = (b + _PF) % _NBUF

            @pl.when(p < _NCHUNK)
            def _():
                @pl.when(p - _NBUF >= 0)
                def _():
                    scatter(p - _NBUF, pb).wait()

                fix(p)
                gather(p, pb).start()

            gather(c, b).wait()
            scatter(c, b).start()
        return carry

    lax.fori_loop(0, _NCHUNK // _NBUF, group, 0)
    for c in range(_NCHUNK - _NBUF, _NCHUNK):
        scatter(c, c % _NBUF).wait()


@jax.jit
def _gather(flat_in, flat_idx):
    mesh = plsc.VectorSubcoreMesh(core_axis_name="c", subcore_axis_name="s")
    return pl.kernel(
        _gather_body,
        out_type=jax.ShapeDtypeStruct((_ROWS, _D), jnp.float32),
        mesh=mesh,
        scratch_types=[
            pltpu.VMEM((_RPW,), jnp.int32),
            [pltpu.VMEM((_CHUNK, _D), jnp.float32) for _ in range(_NBUF)],
            [pltpu.SemaphoreType.DMA for _ in range(_NBUF)],
            [pltpu.SemaphoreType.DMA for _ in range(_NBUF)],
        ],
    )(flat_in, flat_idx)


def kernel(inputs, word_ids):
    flat_in = inputs.reshape(_B * _T, _D)
    out = _gather(flat_in, word_ids.astype(jnp.int32))
    return out.reshape(_B, _L, _D)


# speedup vs baseline: 1.7556x; 1.0009x over previous
import jax
import jax.numpy as jnp
from jax import lax
from jax.experimental import pallas as pl
from jax.experimental.pallas import tpu as pltpu
from jax.experimental.pallas import tpu_sc as plsc

_B, _T, _D = 4, 8192, 768
_L = 8192

_INFO = plsc.get_sparse_core_info()
_NC, _NS, _LANES = _INFO.num_cores, _INFO.num_subcores, _INFO.num_lanes
_NW = _NC * _NS
_ROWS = _B * _L
_RPW = _ROWS // _NW
_WPB = _NW // _B
_CHUNK = 16
_NCHUNK = _RPW // _CHUNK
_NBUF = 8
_PF = 4


def _gather_body(in_hbm, idx_hbm, out_hbm, idx_v, bufs, gsems, ssems):
    wid = lax.axis_index("s") * _NC + lax.axis_index("c")
    base = wid * _RPW
    boff = (wid // _WPB) * _T

    pltpu.sync_copy(
        idx_hbm.at[wid // _WPB, pl.ds((wid % _WPB) * _RPW, _RPW)], idx_v
    )

    _FPC = _LANES // _CHUNK

    def fix(c):
        sl = pl.ds(c * _CHUNK, _LANES)
        idx_v[sl] = jnp.maximum(idx_v[sl], 0) + boff

    def gather(c, b):
        return pltpu.make_async_copy(
            in_hbm.at[idx_v.at[pl.ds(c * _CHUNK, _CHUNK)]], bufs[b], gsems[b]
        )

    def scatter(c, b):
        return pltpu.make_async_copy(
            bufs[b], out_hbm.at[pl.ds(base + c * _CHUNK, _CHUNK)], ssems[b]
        )

    for p in range(_PF):
        if p % _FPC == 0:
            fix(p)
        gather(p, p).start()

    def group(g, carry):
        for b in range(_NBUF):
            c = g * _NBUF + b
            p = c + _PF
            pb = (b + _PF) % _NBUF

            @pl.when(p < _NCHUNK)
            def _():
                @pl.when(p - _NBUF >= 0)
                def _():
                    scatter(p - _NBUF, pb).wait()

                if (b + _PF) % _FPC == 0:
                    fix(p)
                gather(p, pb).start()

            gather(c, b).wait()
            scatter(c, b).start()
        return carry

    lax.fori_loop(0, _NCHUNK // _NBUF, group, 0)
    for c in range(_NCHUNK - _NBUF, _NCHUNK):
        scatter(c, c % _NBUF).wait()


@jax.jit
def _gather(flat_in, flat_idx):
    mesh = plsc.VectorSubcoreMesh(core_axis_name="c", subcore_axis_name="s")
    return pl.kernel(
        _gather_body,
        out_type=jax.ShapeDtypeStruct((_ROWS, _D), jnp.float32),
        mesh=mesh,
        scratch_types=[
            pltpu.VMEM((_RPW,), jnp.int32),
            [pltpu.VMEM((_CHUNK, _D), jnp.float32) for _ in range(_NBUF)],
            [pltpu.SemaphoreType.DMA for _ in range(_NBUF)],
            [pltpu.SemaphoreType.DMA for _ in range(_NBUF)],
        ],
    )(flat_in, flat_idx)


def kernel(inputs, word_ids):
    flat_in = inputs.reshape(_B * _T, _D)
    out = _gather(flat_in, word_ids.astype(jnp.int32))
    return out.reshape(_B, _L, _D)
